# Initial kernel scaffold; baseline (speedup 1.0000x reference)
import jax
import jax.numpy as jnp
from jax.experimental import pallas as pl


def kernel(cond_emb, enc_emb, enc_wih, enc_whh, enc_bih, enc_bhh, mean_w, mean_b, lgv_w, lgv_b, l2h_w, l2h_b, l2c_w, l2c_b, dec_emb, dec_wih, dec_whh, dec_bih, dec_bhh, out_w, out_b, eps, input_ids, target_ids, input_c, target_c):
    raise NotImplementedError("write your pallas kernel here")


# speedup vs baseline: 5.7859x; 5.7859x over previous
import jax
import jax.numpy as jnp
from jax import lax
from jax.experimental import pallas as pl
from jax.experimental.pallas import tpu as pltpu

_H = 1024
_C = 64
_L = 256
_V = 32000
_T = 256
_G = 4 * _H

_F32 = jnp.float32


def _lstm_gates(g, c):
    i = jax.nn.sigmoid(g[:, :_H])
    f = jax.nn.sigmoid(g[:, _H:2 * _H])
    gg = jnp.tanh(g[:, 2 * _H:3 * _H])
    o = jax.nn.sigmoid(g[:, 3 * _H:])
    c = f * c + i * gg
    h = o * jnp.tanh(c)
    return h, c


def _gather_rows(ids_ref, emb_ref, x_vmem, sem):
    for t in range(_T):
        pltpu.make_async_copy(emb_ref.at[ids_ref[t]], x_vmem.at[t], sem).start()
    for t in range(_T):
        pltpu.make_async_copy(emb_ref.at[ids_ref[t]], x_vmem.at[t], sem).wait()


def _enc_kernel(ids_ref, emb_ref, wih_ref, whh_ref, bih_ref, bhh_ref, h0_ref,
                mean_w_ref, mean_b_ref, lgv_w_ref, lgv_b_ref,
                l2h_w_ref, l2h_b_ref, l2c_w_ref, l2c_b_ref,
                eps_ref, cond_t_ref,
                m_ref, lgv_ref, dh0_ref, dc0_ref,
                x_vmem, xproj, sem):
    _gather_rows(ids_ref, emb_ref, x_vmem, sem)
    b = bih_ref[...] + bhh_ref[...]
    xp = jnp.dot(x_vmem[...], wih_ref[...], preferred_element_type=_F32) + b
    xproj[...] = xp.reshape(_T, 1, _G)

    def body(t, carry):
        h, c = carry
        g = xproj[t] + jnp.dot(h, whh_ref[...], preferred_element_type=_F32)
        return _lstm_gates(g, c)

    h0 = h0_ref[...]
    c0 = jnp.zeros((1, _H), _F32)
    h, _ = lax.fori_loop(0, _T, body, (h0, c0))

    m = jnp.dot(h, mean_w_ref[...], preferred_element_type=_F32) + mean_b_ref[...]
    lgv = jnp.dot(h, lgv_w_ref[...], preferred_element_type=_F32) + lgv_b_ref[...]
    z = eps_ref[...] * jnp.exp(lgv * 0.5) + m
    zc = jnp.concatenate([z, cond_t_ref[...]], axis=1)
    dh0_ref[...] = jnp.dot(zc, l2h_w_ref[...], preferred_element_type=_F32) + l2h_b_ref[...]
    dc0_ref[...] = jnp.dot(zc, l2c_w_ref[...], preferred_element_type=_F32) + l2c_b_ref[...]
    m_ref[...] = m
    lgv_ref[...] = lgv


def _dec_kernel(ids_ref, emb_ref, wih_ref, whh_ref, bih_ref, bhh_ref,
                dh0_ref, dc0_ref,
                hs_ref,
                x_vmem, xproj, sem):
    _gather_rows(ids_ref, emb_ref, x_vmem, sem)
    b = bih_ref[...] + bhh_ref[...]
    xp = jnp.dot(x_vmem[...], wih_ref[...], preferred_element_type=_F32) + b
    xproj[...] = xp.reshape(_T, 1, _G)

    def body(t, carry):
        h, c = carry
        g = xproj[t] + jnp.dot(h, whh_ref[...], preferred_element_type=_F32)
        h, c = _lstm_gates(g, c)
        hs_ref[t] = h
        return (h, c)

    lax.fori_loop(0, _T, body, (dh0_ref[...], dc0_ref[...]))


def _out_kernel(hs_ref, w_ref, b_ref, o_ref):
    o_ref[...] = lax.dot_general(
        hs_ref[...], w_ref[...],
        dimension_numbers=(((1,), (1,)), ((), ())),
        preferred_element_type=_F32) + b_ref[...]


_VB = 1280


def _run(interpret, cond_emb, enc_emb, enc_wih, enc_whh, enc_bih, enc_bhh,
         mean_w, mean_b, lgv_w, lgv_b, l2h_w, l2h_b, l2c_w, l2c_b,
         dec_emb, dec_wih, dec_whh, dec_bih, dec_bhh, out_w, out_b,
         eps, input_ids, target_ids, input_c, target_c):
    ids_e = input_ids.astype(jnp.int32)
    ids_d = jnp.concatenate(
        [jnp.zeros((1,), jnp.int32), target_ids[:-1].astype(jnp.int32)])
    cond_i = cond_emb[input_c]
    cond_t = cond_emb[target_c].reshape(1, _C)
    h0 = jnp.concatenate([jnp.zeros((_H - _C,), _F32), cond_i]).reshape(1, _H)

    smem = pl.BlockSpec(memory_space=pltpu.SMEM)
    anyspace = pl.BlockSpec(memory_space=pl.ANY)
    vmem = pl.BlockSpec(memory_space=pltpu.VMEM)

    enc_out = pl.pallas_call(
        _enc_kernel,
        out_shape=[
            jax.ShapeDtypeStruct((1, _L), _F32),
            jax.ShapeDtypeStruct((1, _L), _F32),
            jax.ShapeDtypeStruct((1, _H), _F32),
            jax.ShapeDtypeStruct((1, _H), _F32),
        ],
        in_specs=[smem, anyspace] + [vmem] * 15,
        out_specs=[vmem] * 4,
        scratch_shapes=[
            pltpu.VMEM((_T, _H), _F32),
            pltpu.VMEM((_T, 1, _G), _F32),
            pltpu.SemaphoreType.DMA,
        ],
        compiler_params=pltpu.CompilerParams(
            vmem_limit_bytes=56 * 1024 * 1024),
        name="enc_lstm_vae",
        interpret=interpret,
    )(ids_e, enc_emb, enc_wih.T, enc_whh.T,
      enc_bih.reshape(1, _G), enc_bhh.reshape(1, _G), h0,
      mean_w.T, mean_b.reshape(1, _L), lgv_w.T, lgv_b.reshape(1, _L),
      l2h_w.T, l2h_b.reshape(1, _H), l2c_w.T, l2c_b.reshape(1, _H),
      eps.reshape(1, _L), cond_t)
    m, lgv, dh0, dc0 = enc_out

    hs = pl.pallas_call(
        _dec_kernel,
        out_shape=jax.ShapeDtypeStruct((_T, 1, _H), _F32),
        in_specs=[smem, anyspace] + [vmem] * 6,
        out_specs=vmem,
        scratch_shapes=[
            pltpu.VMEM((_T, _H), _F32),
            pltpu.VMEM((_T, 1, _G), _F32),
            pltpu.SemaphoreType.DMA,
        ],
        compiler_params=pltpu.CompilerParams(
            vmem_limit_bytes=56 * 1024 * 1024),
        name="dec_lstm",
        interpret=interpret,
    )(ids_d, dec_emb, dec_wih.T, dec_whh.T,
      dec_bih.reshape(1, _G), dec_bhh.reshape(1, _G), dh0, dc0)
    hs2 = hs.reshape(_T, _H)

    logits = pl.pallas_call(
        _out_kernel,
        out_shape=jax.ShapeDtypeStruct((_T, _V), _F32),
        grid=(_V // _VB,),
        in_specs=[
            pl.BlockSpec((_T, _H), lambda i: (0, 0)),
            pl.BlockSpec((_VB, _H), lambda i: (i, 0)),
            pl.BlockSpec((1, _VB), lambda i: (0, i)),
        ],
        out_specs=pl.BlockSpec((_T, _VB), lambda i: (0, i)),
        compiler_params=pltpu.CompilerParams(
            dimension_semantics=("parallel",)),
        name="out_proj",
        interpret=interpret,
    )(hs2, out_w, out_b.reshape(1, _V))

    return logits, m.reshape(_L), lgv.reshape(_L)


def kernel(cond_emb, enc_emb, enc_wih, enc_whh, enc_bih, enc_bhh, mean_w,
           mean_b, lgv_w, lgv_b, l2h_w, l2h_b, l2c_w, l2c_b, dec_emb,
           dec_wih, dec_whh, dec_bih, dec_bhh, out_w, out_b, eps, input_ids,
           target_ids, input_c, target_c):
    return _run(False, cond_emb, enc_emb, enc_wih, enc_whh, enc_bih, enc_bhh,
                mean_w, mean_b, lgv_w, lgv_b, l2h_w, l2h_b, l2c_w, l2c_b,
                dec_emb, dec_wih, dec_whh, dec_bih, dec_bhh, out_w, out_b,
                eps, input_ids, target_ids, input_c, target_c)


# speedup vs baseline: 6.1716x; 1.0667x over previous
import jax
import jax.numpy as jnp
from jax import lax
from jax.experimental import pallas as pl
from jax.experimental.pallas import tpu as pltpu

_H = 1024
_C = 64
_L = 256
_V = 32000
_T = 256
_G = 4 * _H

_F32 = jnp.float32
_BF16 = jnp.bfloat16

_DNT = (((1,), (1,)), ((), ()))


def _lstm_gates(g, c):
    i = jax.nn.sigmoid(g[:, :_H])
    f = jax.nn.sigmoid(g[:, _H:2 * _H])
    gg = jnp.tanh(g[:, 2 * _H:3 * _H])
    o = jax.nn.sigmoid(g[:, 3 * _H:])
    c = f * c + i * gg
    h = o * jnp.tanh(c)
    return h, c


def _gather_rows(ids_ref, emb_ref, x_vmem, sem):
    for t in range(_T):
        pltpu.make_async_copy(emb_ref.at[ids_ref[t]], x_vmem.at[t], sem).start()
    for t in range(_T):
        pltpu.make_async_copy(emb_ref.at[ids_ref[t]], x_vmem.at[t], sem).wait()


def _xproj(x_vmem, wih_ref, bih_ref, bhh_ref, xproj):
    b = bih_ref[...] + bhh_ref[...]
    xp = lax.dot_general(x_vmem[...].astype(_BF16), wih_ref[...].astype(_BF16),
                         _DNT, preferred_element_type=_F32) + b
    xproj[...] = xp.reshape(_T, 1, _G)


def _enc_kernel(ids_ref, emb_ref, wih_ref, whh_ref, bih_ref, bhh_ref, h0_ref,
                mean_w_ref, mean_b_ref, lgv_w_ref, lgv_b_ref,
                l2h_w_ref, l2h_b_ref, l2c_w_ref, l2c_b_ref,
                eps_ref, cond_t_ref,
                m_ref, lgv_ref, dh0_ref, dc0_ref,
                x_vmem, xproj, whh_t, sem):
    _gather_rows(ids_ref, emb_ref, x_vmem, sem)
    whh_t[...] = whh_ref[...].T.astype(_BF16)
    _xproj(x_vmem, wih_ref, bih_ref, bhh_ref, xproj)

    def body(t, carry):
        h, c = carry
        g = xproj[t] + jnp.dot(h.astype(_BF16), whh_t[...],
                               preferred_element_type=_F32)
        return _lstm_gates(g, c)

    h0 = h0_ref[...]
    c0 = jnp.zeros((1, _H), _F32)
    h, _ = lax.fori_loop(0, _T, body, (h0, c0))

    m = lax.dot_general(h, mean_w_ref[...], _DNT,
                        preferred_element_type=_F32) + mean_b_ref[...]
    lgv = lax.dot_general(h, lgv_w_ref[...], _DNT,
                          preferred_element_type=_F32) + lgv_b_ref[...]
    z = eps_ref[...] * jnp.exp(lgv * 0.5) + m
    zc = jnp.concatenate([z, cond_t_ref[...]], axis=1)
    dh0_ref[...] = lax.dot_general(zc, l2h_w_ref[...], _DNT,
                                   preferred_element_type=_F32) + l2h_b_ref[...]
    dc0_ref[...] = lax.dot_general(zc, l2c_w_ref[...], _DNT,
                                   preferred_element_type=_F32) + l2c_b_ref[...]
    m_ref[...] = m
    lgv_ref[...] = lgv


def _dec_kernel(ids_ref, emb_ref, wih_ref, whh_ref, bih_ref, bhh_ref,
                dh0_ref, dc0_ref,
                hs_ref,
                x_vmem, xproj, whh_t, sem):
    _gather_rows(ids_ref, emb_ref, x_vmem, sem)
    whh_t[...] = whh_ref[...].T.astype(_BF16)
    _xproj(x_vmem, wih_ref, bih_ref, bhh_ref, xproj)

    def body(t, carry):
        h, c = carry
        g = xproj[t] + jnp.dot(h.astype(_BF16), whh_t[...],
                               preferred_element_type=_F32)
        h, c = _lstm_gates(g, c)
        hs_ref[t] = h
        return (h, c)

    lax.fori_loop(0, _T, body, (dh0_ref[...], dc0_ref[...]))


def _out_kernel(hs_ref, w_ref, b_ref, o_ref):
    o_ref[...] = lax.dot_general(
        hs_ref[...], w_ref[...], _DNT,
        preferred_element_type=_F32) + b_ref[...]


_VB = 1280


def kernel(cond_emb, enc_emb, enc_wih, enc_whh, enc_bih, enc_bhh, mean_w,
           mean_b, lgv_w, lgv_b, l2h_w, l2h_b, l2c_w, l2c_b, dec_emb,
           dec_wih, dec_whh, dec_bih, dec_bhh, out_w, out_b, eps, input_ids,
           target_ids, input_c, target_c, *, interpret=False):
    ids_e = input_ids.astype(jnp.int32)
    ids_d = jnp.concatenate(
        [jnp.zeros((1,), jnp.int32), target_ids[:-1].astype(jnp.int32)])
    cond_i = cond_emb[input_c]
    cond_t = cond_emb[target_c].reshape(1, _C)
    h0 = jnp.concatenate([jnp.zeros((_H - _C,), _F32), cond_i]).reshape(1, _H)

    smem = pl.BlockSpec(memory_space=pltpu.SMEM)
    anyspace = pl.BlockSpec(memory_space=pl.ANY)
    vmem = pl.BlockSpec(memory_space=pltpu.VMEM)

    enc_out = pl.pallas_call(
        _enc_kernel,
        out_shape=[
            jax.ShapeDtypeStruct((1, _L), _F32),
            jax.ShapeDtypeStruct((1, _L), _F32),
            jax.ShapeDtypeStruct((1, _H), _F32),
            jax.ShapeDtypeStruct((1, _H), _F32),
        ],
        in_specs=[smem, anyspace] + [vmem] * 15,
        out_specs=[vmem] * 4,
        scratch_shapes=[
            pltpu.VMEM((_T, _H), _F32),
            pltpu.VMEM((_T, 1, _G), _F32),
            pltpu.VMEM((_H, _G), _BF16),
            pltpu.SemaphoreType.DMA,
        ],
        compiler_params=pltpu.CompilerParams(
            vmem_limit_bytes=56 * 1024 * 1024),
        name="enc_lstm_vae",
        interpret=interpret,
    )(ids_e, enc_emb, enc_wih, enc_whh,
      enc_bih.reshape(1, _G), enc_bhh.reshape(1, _G), h0,
      mean_w, mean_b.reshape(1, _L), lgv_w, lgv_b.reshape(1, _L),
      l2h_w, l2h_b.reshape(1, _H), l2c_w, l2c_b.reshape(1, _H),
      eps.reshape(1, _L), cond_t)
    m, lgv, dh0, dc0 = enc_out

    hs = pl.pallas_call(
        _dec_kernel,
        out_shape=jax.ShapeDtypeStruct((_T, 1, _H), _F32),
        in_specs=[smem, anyspace] + [vmem] * 6,
        out_specs=vmem,
        scratch_shapes=[
            pltpu.VMEM((_T, _H), _F32),
            pltpu.VMEM((_T, 1, _G), _F32),
            pltpu.VMEM((_H, _G), _BF16),
            pltpu.SemaphoreType.DMA,
        ],
        compiler_params=pltpu.CompilerParams(
            vmem_limit_bytes=56 * 1024 * 1024),
        name="dec_lstm",
        interpret=interpret,
    )(ids_d, dec_emb, dec_wih, dec_whh,
      dec_bih.reshape(1, _G), dec_bhh.reshape(1, _G), dh0, dc0)
    hs2 = hs.reshape(_T, _H)

    logits = pl.pallas_call(
        _out_kernel,
        out_shape=jax.ShapeDtypeStruct((_T, _V), _F32),
        grid=(_V // _VB,),
        in_specs=[
            pl.BlockSpec((_T, _H), lambda i: (0, 0)),
            pl.BlockSpec((_VB, _H), lambda i: (i, 0)),
            pl.BlockSpec((1, _VB), lambda i: (0, i)),
        ],
        out_specs=pl.BlockSpec((_T, _VB), lambda i: (0, i)),
        compiler_params=pltpu.CompilerParams(
            dimension_semantics=("parallel",)),
        name="out_proj",
        interpret=interpret,
    )(hs2, out_w, out_b.reshape(1, _V))

    return logits, m.reshape(_L), lgv.reshape(_L)


# speedup vs baseline: 6.2645x; 1.0151x over previous
import jax
import jax.numpy as jnp
from jax import lax
from jax.experimental import pallas as pl
from jax.experimental.pallas import tpu as pltpu

_H = 1024
_C = 64
_L = 256
_V = 32000
_T = 256
_G = 4 * _H

_F32 = jnp.float32
_BF16 = jnp.bfloat16

_DNT = (((1,), (1,)), ((), ()))


def _lstm_gates(g, c):
    i = jax.nn.sigmoid(g[:, :_H])
    f = jax.nn.sigmoid(g[:, _H:2 * _H])
    gg = jnp.tanh(g[:, 2 * _H:3 * _H])
    o = jax.nn.sigmoid(g[:, 3 * _H:])
    c = f * c + i * gg
    h = o * jnp.tanh(c)
    return h, c


def _gather_rows(ids_ref, emb_ref, x_vmem, sem):
    for t in range(_T):
        pltpu.make_async_copy(emb_ref.at[ids_ref[t]], x_vmem.at[t], sem).start()
    for t in range(_T):
        pltpu.make_async_copy(emb_ref.at[ids_ref[t]], x_vmem.at[t], sem).wait()


def _xproj(x_vmem, wih_ref, bih_ref, bhh_ref, xproj):
    b = bih_ref[...] + bhh_ref[...]
    xp = lax.dot_general(x_vmem[...].astype(_BF16), wih_ref[...].astype(_BF16),
                         _DNT, preferred_element_type=_F32) + b
    xproj[...] = xp.reshape(_T, 1, _G)


def _enc_kernel(ids_ref, emb_ref, wih_ref, whh_ref, bih_ref, bhh_ref, h0_ref,
                mean_w_ref, mean_b_ref, lgv_w_ref, lgv_b_ref,
                l2h_w_ref, l2h_b_ref, l2c_w_ref, l2c_b_ref,
                eps_ref, cond_t_ref,
                m_ref, lgv_ref, dh0_ref, dc0_ref,
                x_vmem, xproj, whh_t, sem):
    _gather_rows(ids_ref, emb_ref, x_vmem, sem)
    for j in range(8):
        sl = slice(j * 512, (j + 1) * 512)
        whh_t[:, sl] = whh_ref[sl, :].T.astype(_BF16)
    _xproj(x_vmem, wih_ref, bih_ref, bhh_ref, xproj)

    def body(i, carry):
        h, c = carry
        t = i * 2
        w = whh_t[...]
        for s in range(2):
            g = xproj[t + s] + jnp.dot(h.astype(_BF16), w,
                                       preferred_element_type=_F32)
            h, c = _lstm_gates(g, c)
        return (h, c)

    h0 = h0_ref[...]
    c0 = jnp.zeros((1, _H), _F32)
    h, _ = lax.fori_loop(0, _T // 2, body, (h0, c0))

    m = lax.dot_general(h, mean_w_ref[...], _DNT,
                        preferred_element_type=_F32) + mean_b_ref[...]
    lgv = lax.dot_general(h, lgv_w_ref[...], _DNT,
                          preferred_element_type=_F32) + lgv_b_ref[...]
    z = eps_ref[...] * jnp.exp(lgv * 0.5) + m
    zc = jnp.concatenate([z, cond_t_ref[...]], axis=1)
    dh0_ref[...] = lax.dot_general(zc, l2h_w_ref[...], _DNT,
                                   preferred_element_type=_F32) + l2h_b_ref[...]
    dc0_ref[...] = lax.dot_general(zc, l2c_w_ref[...], _DNT,
                                   preferred_element_type=_F32) + l2c_b_ref[...]
    m_ref[...] = m
    lgv_ref[...] = lgv


def _dec_kernel(ids_ref, emb_ref, wih_ref, whh_ref, bih_ref, bhh_ref,
                dh0_ref, dc0_ref,
                hs_ref,
                x_vmem, xproj, whh_t, sem):
    _gather_rows(ids_ref, emb_ref, x_vmem, sem)
    for j in range(8):
        sl = slice(j * 512, (j + 1) * 512)
        whh_t[:, sl] = whh_ref[sl, :].T.astype(_BF16)
    _xproj(x_vmem, wih_ref, bih_ref, bhh_ref, xproj)

    def body(i, carry):
        h, c = carry
        t = i * 2
        w = whh_t[...]
        for s in range(2):
            g = xproj[t + s] + jnp.dot(h.astype(_BF16), w,
                                       preferred_element_type=_F32)
            h, c = _lstm_gates(g, c)
            hs_ref[t + s] = h
        return (h, c)

    lax.fori_loop(0, _T // 2, body, (dh0_ref[...], dc0_ref[...]))


def _out_kernel(hs_ref, w_ref, b_ref, o_ref):
    o_ref[...] = lax.dot_general(
        hs_ref[...], w_ref[...], _DNT,
        preferred_element_type=_F32) + b_ref[...]


_VB = 3200


def kernel(cond_emb, enc_emb, enc_wih, enc_whh, enc_bih, enc_bhh, mean_w,
           mean_b, lgv_w, lgv_b, l2h_w, l2h_b, l2c_w, l2c_b, dec_emb,
           dec_wih, dec_whh, dec_bih, dec_bhh, out_w, out_b, eps, input_ids,
           target_ids, input_c, target_c, *, interpret=False):
    ids_e = input_ids.astype(jnp.int32)
    ids_d = jnp.concatenate(
        [jnp.zeros((1,), jnp.int32), target_ids[:-1].astype(jnp.int32)])
    cond_i = cond_emb[input_c]
    cond_t = cond_emb[target_c].reshape(1, _C)
    h0 = jnp.concatenate([jnp.zeros((_H - _C,), _F32), cond_i]).reshape(1, _H)

    smem = pl.BlockSpec(memory_space=pltpu.SMEM)
    anyspace = pl.BlockSpec(memory_space=pl.ANY)
    vmem = pl.BlockSpec(memory_space=pltpu.VMEM)

    enc_out = pl.pallas_call(
        _enc_kernel,
        out_shape=[
            jax.ShapeDtypeStruct((1, _L), _F32),
            jax.ShapeDtypeStruct((1, _L), _F32),
            jax.ShapeDtypeStruct((1, _H), _F32),
            jax.ShapeDtypeStruct((1, _H), _F32),
        ],
        in_specs=[smem, anyspace] + [vmem] * 15,
        out_specs=[vmem] * 4,
        scratch_shapes=[
            pltpu.VMEM((_T, _H), _F32),
            pltpu.VMEM((_T, 1, _G), _F32),
            pltpu.VMEM((_H, _G), _BF16),
            pltpu.SemaphoreType.DMA,
        ],
        compiler_params=pltpu.CompilerParams(
            vmem_limit_bytes=61 * 1024 * 1024),
        name="enc_lstm_vae",
        interpret=interpret,
    )(ids_e, enc_emb, enc_wih, enc_whh,
      enc_bih.reshape(1, _G), enc_bhh.reshape(1, _G), h0,
      mean_w, mean_b.reshape(1, _L), lgv_w, lgv_b.reshape(1, _L),
      l2h_w, l2h_b.reshape(1, _H), l2c_w, l2c_b.reshape(1, _H),
      eps.reshape(1, _L), cond_t)
    m, lgv, dh0, dc0 = enc_out

    hs = pl.pallas_call(
        _dec_kernel,
        out_shape=jax.ShapeDtypeStruct((_T, 1, _H), _F32),
        in_specs=[smem, anyspace] + [vmem] * 6,
        out_specs=vmem,
        scratch_shapes=[
            pltpu.VMEM((_T, _H), _F32),
            pltpu.VMEM((_T, 1, _G), _F32),
            pltpu.VMEM((_H, _G), _BF16),
            pltpu.SemaphoreType.DMA,
        ],
        compiler_params=pltpu.CompilerParams(
            vmem_limit_bytes=61 * 1024 * 1024),
        name="dec_lstm",
        interpret=interpret,
    )(ids_d, dec_emb, dec_wih, dec_whh,
      dec_bih.reshape(1, _G), dec_bhh.reshape(1, _G), dh0, dc0)
    hs2 = hs.reshape(_T, _H)

    logits = pl.pallas_call(
        _out_kernel,
        out_shape=jax.ShapeDtypeStruct((_T, _V), _F32),
        grid=(_V // _VB,),
        in_specs=[
            pl.BlockSpec((_T, _H), lambda i: (0, 0)),
            pl.BlockSpec((_VB, _H), lambda i: (i, 0)),
            pl.BlockSpec((1, _VB), lambda i: (0, i)),
        ],
        out_specs=pl.BlockSpec((_T, _VB), lambda i: (0, i)),
        compiler_params=pltpu.CompilerParams(
            dimension_semantics=("parallel",),
            vmem_limit_bytes=56 * 1024 * 1024),
        name="out_proj",
        interpret=interpret,
    )(hs2, out_w, out_b.reshape(1, _V))

    return logits, m.reshape(_L), lgv.reshape(_L)


# speedup vs baseline: 6.3417x; 1.0123x over previous
import jax
import jax.numpy as jnp
from jax import lax
from jax.experimental import pallas as pl
from jax.experimental.pallas import tpu as pltpu

_H = 1024
_C = 64
_L = 256
_V = 32000
_T = 256
_G = 4 * _H

_F32 = jnp.float32
_BF16 = jnp.bfloat16

_DNT = (((1,), (1,)), ((), ()))


def _lstm_gates(g, c):
    i = jax.nn.sigmoid(g[:, :_H])
    f = jax.nn.sigmoid(g[:, _H:2 * _H])
    gg = jnp.tanh(g[:, 2 * _H:3 * _H])
    o = jax.nn.sigmoid(g[:, 3 * _H:])
    c = f * c + i * gg
    h = o * jnp.tanh(c)
    return h, c


def _start_gather(ids_ref, emb_ref, x_vmem, sem):
    for t in range(_T):
        pltpu.make_async_copy(emb_ref.at[ids_ref[t]], x_vmem.at[t], sem).start()


def _wait_gather(ids_ref, emb_ref, x_vmem, sem):
    for t in range(_T):
        pltpu.make_async_copy(emb_ref.at[ids_ref[t]], x_vmem.at[t], sem).wait()


def _transpose_whh(whh_buf, whh_t):
    for j in range(8):
        sl = slice(j * 512, (j + 1) * 512)
        whh_t[:, sl] = whh_buf[sl, :].T.astype(_BF16)


def _xproj(x_vmem, wih_buf, bih_ref, bhh_ref, xproj):
    b = bih_ref[...] + bhh_ref[...]
    xp = lax.dot_general(x_vmem[...].astype(_BF16), wih_buf[...].astype(_BF16),
                         _DNT, preferred_element_type=_F32) + b
    xproj[...] = xp.reshape(_T, 1, _G)


def _recurrence(xproj, whh_t, h, c, hs_ref=None):
    def body(i, carry):
        h, c = carry
        t = i * 2
        w = whh_t[...]
        for s in range(2):
            g = xproj[t + s] + jnp.dot(h.astype(_BF16), w,
                                       preferred_element_type=_F32)
            h, c = _lstm_gates(g, c)
            if hs_ref is not None:
                hs_ref[t + s] = h
        return (h, c)

    return lax.fori_loop(0, _T // 2, body, (h, c))


def _seq_kernel(ids_e_ref, ids_d_ref, enc_emb, dec_emb,
                enc_wih, enc_whh, dec_wih, dec_whh,
                ebih_ref, ebhh_ref, dbih_ref, dbhh_ref, h0_ref,
                mean_w_ref, mean_b_ref, lgv_w_ref, lgv_b_ref,
                l2h_w_ref, l2h_b_ref, l2c_w_ref, l2c_b_ref,
                eps_ref, cond_t_ref,
                hs_ref, m_ref, lgv_ref,
                x_vmem, xproj, whh_t, wih_buf, whh_buf,
                sem_g, sem_w, sem_h):
    wdma = pltpu.make_async_copy(enc_wih, wih_buf, sem_w)
    wdma.start()
    hdma = pltpu.make_async_copy(enc_whh, whh_buf, sem_h)
    hdma.start()
    _start_gather(ids_e_ref, enc_emb, x_vmem, sem_g)
    hdma.wait()
    _transpose_whh(whh_buf, whh_t)
    _wait_gather(ids_e_ref, enc_emb, x_vmem, sem_g)
    wdma.wait()
    _xproj(x_vmem, wih_buf, ebih_ref, ebhh_ref, xproj)

    wdma2 = pltpu.make_async_copy(dec_wih, wih_buf, sem_w)
    wdma2.start()
    hdma2 = pltpu.make_async_copy(dec_whh, whh_buf, sem_h)
    hdma2.start()
    _start_gather(ids_d_ref, dec_emb, x_vmem, sem_g)

    h0 = h0_ref[...]
    c0 = jnp.zeros((1, _H), _F32)
    h, _ = _recurrence(xproj, whh_t, h0, c0)

    m = lax.dot_general(h, mean_w_ref[...], _DNT,
                        preferred_element_type=_F32) + mean_b_ref[...]
    lgv = lax.dot_general(h, lgv_w_ref[...], _DNT,
                          preferred_element_type=_F32) + lgv_b_ref[...]
    z = eps_ref[...] * jnp.exp(lgv * 0.5) + m
    zc = jnp.concatenate([z, cond_t_ref[...]], axis=1)
    dh0 = lax.dot_general(zc, l2h_w_ref[...], _DNT,
                          preferred_element_type=_F32) + l2h_b_ref[...]
    dc0 = lax.dot_general(zc, l2c_w_ref[...], _DNT,
                          preferred_element_type=_F32) + l2c_b_ref[...]
    m_ref[...] = m
    lgv_ref[...] = lgv

    hdma2.wait()
    _transpose_whh(whh_buf, whh_t)
    _wait_gather(ids_d_ref, dec_emb, x_vmem, sem_g)
    wdma2.wait()
    _xproj(x_vmem, wih_buf, dbih_ref, dbhh_ref, xproj)
    _recurrence(xproj, whh_t, dh0, dc0, hs_ref)


def _out_kernel(hs_ref, w_ref, b_ref, o_ref):
    o_ref[...] = lax.dot_general(
        hs_ref[...], w_ref[...], _DNT,
        preferred_element_type=_F32) + b_ref[...]


_VB = 3200


def kernel(cond_emb, enc_emb, enc_wih, enc_whh, enc_bih, enc_bhh, mean_w,
           mean_b, lgv_w, lgv_b, l2h_w, l2h_b, l2c_w, l2c_b, dec_emb,
           dec_wih, dec_whh, dec_bih, dec_bhh, out_w, out_b, eps, input_ids,
           target_ids, input_c, target_c, *, interpret=False):
    ids_e = input_ids.astype(jnp.int32)
    ids_d = jnp.concatenate(
        [jnp.zeros((1,), jnp.int32), target_ids[:-1].astype(jnp.int32)])
    cond_i = cond_emb[input_c]
    cond_t = cond_emb[target_c].reshape(1, _C)
    h0 = jnp.concatenate([jnp.zeros((_H - _C,), _F32), cond_i]).reshape(1, _H)

    smem = pl.BlockSpec(memory_space=pltpu.SMEM)
    anyspace = pl.BlockSpec(memory_space=pl.ANY)
    vmem = pl.BlockSpec(memory_space=pltpu.VMEM)

    hs, m, lgv = pl.pallas_call(
        _seq_kernel,
        out_shape=[
            jax.ShapeDtypeStruct((_T, 1, _H), _F32),
            jax.ShapeDtypeStruct((1, _L), _F32),
            jax.ShapeDtypeStruct((1, _L), _F32),
        ],
        in_specs=[smem, smem] + [anyspace] * 6 + [vmem] * 15,
        out_specs=[vmem] * 3,
        scratch_shapes=[
            pltpu.VMEM((_T, _H), _F32),
            pltpu.VMEM((_T, 1, _G), _F32),
            pltpu.VMEM((_H, _G), _BF16),
            pltpu.VMEM((_G, _H), _F32),
            pltpu.VMEM((_G, _H), _F32),
            pltpu.SemaphoreType.DMA,
            pltpu.SemaphoreType.DMA,
            pltpu.SemaphoreType.DMA,
        ],
        compiler_params=pltpu.CompilerParams(
            vmem_limit_bytes=61 * 1024 * 1024),
        name="seq_lstm",
        interpret=interpret,
    )(ids_e, ids_d, enc_emb, dec_emb, enc_wih, enc_whh, dec_wih, dec_whh,
      enc_bih.reshape(1, _G), enc_bhh.reshape(1, _G),
      dec_bih.reshape(1, _G), dec_bhh.reshape(1, _G), h0,
      mean_w, mean_b.reshape(1, _L), lgv_w, lgv_b.reshape(1, _L),
      l2h_w, l2h_b.reshape(1, _H), l2c_w, l2c_b.reshape(1, _H),
      eps.reshape(1, _L), cond_t)
    hs2 = hs.reshape(_T, _H)

    logits = pl.pallas_call(
        _out_kernel,
        out_shape=jax.ShapeDtypeStruct((_T, _V), _F32),
        grid=(_V // _VB,),
        in_specs=[
            pl.BlockSpec((_T, _H), lambda i: (0, 0)),
            pl.BlockSpec((_VB, _H), lambda i: (i, 0)),
            pl.BlockSpec((1, _VB), lambda i: (0, i)),
        ],
        out_specs=pl.BlockSpec((_T, _VB), lambda i: (0, i)),
        compiler_params=pltpu.CompilerParams(
            dimension_semantics=("parallel",),
            vmem_limit_bytes=56 * 1024 * 1024),
        name="out_proj",
        interpret=interpret,
    )(hs2, out_w, out_b.reshape(1, _V))

    return logits, m.reshape(_L), lgv.reshape(_L)


# speedup vs baseline: 7.2237x; 1.1391x over previous
import jax
import jax.numpy as jnp
from jax import lax
from jax.experimental import pallas as pl
from jax.experimental.pallas import tpu as pltpu

_H = 1024
_C = 64
_L = 256
_V = 32000
_T = 256
_G = 4 * _H

_F32 = jnp.float32
_BF16 = jnp.bfloat16

_DNT = (((1,), (1,)), ((), ()))


def _lstm_gates(g_ifg, g_o, c):
    i = jax.nn.sigmoid(g_ifg[:, :_H])
    f = jax.nn.sigmoid(g_ifg[:, _H:2 * _H])
    gg = jnp.tanh(g_ifg[:, 2 * _H:])
    o = jax.nn.sigmoid(g_o)
    c = f * c + i * gg
    h = o * jnp.tanh(c)
    return h, c


def _start_gather(ids_ref, emb_ref, x_vmem, sem):
    for t in range(_T):
        pltpu.make_async_copy(emb_ref.at[ids_ref[t]], x_vmem.at[t], sem).start()


def _wait_gather(ids_ref, emb_ref, x_vmem, sem):
    for t in range(_T):
        pltpu.make_async_copy(emb_ref.at[ids_ref[t]], x_vmem.at[t], sem).wait()


def _transpose_whh(whh_buf, whh_t, whh_tv):
    for j in range(6):
        sl = slice(j * 512, (j + 1) * 512)
        whh_t[:, sl] = whh_buf[sl, :].T.astype(_BF16)
    for j in range(2):
        src = slice(3 * _H + j * 512, 3 * _H + (j + 1) * 512)
        whh_tv[:, j * 512:(j + 1) * 512] = whh_buf[src, :].T


def _xproj(x_vmem, wih_buf, bih_ref, bhh_ref, xproj):
    b = bih_ref[...] + bhh_ref[...]
    xp = lax.dot_general(x_vmem[...].astype(_BF16), wih_buf[...].astype(_BF16),
                         _DNT, preferred_element_type=_F32) + b
    xproj[...] = xp.reshape(_T, 1, _G)


def _recurrence(xproj, whh_t, whh_tv, h, c, hs_ref=None):
    def body(i, carry):
        h, c = carry
        t = i * 2
        w = whh_t[...]
        wv = whh_tv[...]
        for s in range(2):
            xg = xproj[t + s]
            g_ifg = xg[:, :3 * _H] + jnp.dot(h.astype(_BF16), w,
                                             preferred_element_type=_F32)
            h_col = h.reshape(1, _H).T
            g_o = xg[:, 3 * _H:] + jnp.sum(h_col * wv, axis=0, keepdims=True)
            h, c = _lstm_gates(g_ifg, g_o, c)
            if hs_ref is not None:
                hs_ref[t + s] = h
        return (h, c)

    return lax.fori_loop(0, _T // 2, body, (h, c))


def _seq_kernel(ids_e_ref, ids_d_ref, enc_emb, dec_emb,
                enc_wih, enc_whh, dec_wih, dec_whh,
                ebih_ref, ebhh_ref, dbih_ref, dbhh_ref, h0_ref,
                mean_w_ref, mean_b_ref, lgv_w_ref, lgv_b_ref,
                l2h_w_ref, l2h_b_ref, l2c_w_ref, l2c_b_ref,
                eps_ref, cond_t_ref,
                hs_ref, m_ref, lgv_ref,
                x_vmem, xproj, whh_t, whh_tv, wih_buf, whh_buf,
                sem_g, sem_w, sem_h):
    wdma = pltpu.make_async_copy(enc_wih, wih_buf, sem_w)
    wdma.start()
    hdma = pltpu.make_async_copy(enc_whh, whh_buf, sem_h)
    hdma.start()
    _start_gather(ids_e_ref, enc_emb, x_vmem, sem_g)
    hdma.wait()
    _transpose_whh(whh_buf, whh_t, whh_tv)
    _wait_gather(ids_e_ref, enc_emb, x_vmem, sem_g)
    wdma.wait()
    _xproj(x_vmem, wih_buf, ebih_ref, ebhh_ref, xproj)

    wdma2 = pltpu.make_async_copy(dec_wih, wih_buf, sem_w)
    wdma2.start()
    hdma2 = pltpu.make_async_copy(dec_whh, whh_buf, sem_h)
    hdma2.start()
    _start_gather(ids_d_ref, dec_emb, x_vmem, sem_g)

    h0 = h0_ref[...]
    c0 = jnp.zeros((1, _H), _F32)
    h, _ = _recurrence(xproj, whh_t, whh_tv, h0, c0)

    m = lax.dot_general(h, mean_w_ref[...], _DNT,
                        preferred_element_type=_F32) + mean_b_ref[...]
    lgv = lax.dot_general(h, lgv_w_ref[...], _DNT,
                          preferred_element_type=_F32) + lgv_b_ref[...]
    z = eps_ref[...] * jnp.exp(lgv * 0.5) + m
    zc = jnp.concatenate([z, cond_t_ref[...]], axis=1)
    dh0 = lax.dot_general(zc, l2h_w_ref[...], _DNT,
                          preferred_element_type=_F32) + l2h_b_ref[...]
    dc0 = lax.dot_general(zc, l2c_w_ref[...], _DNT,
                          preferred_element_type=_F32) + l2c_b_ref[...]
    m_ref[...] = m
    lgv_ref[...] = lgv

    hdma2.wait()
    _transpose_whh(whh_buf, whh_t, whh_tv)
    _wait_gather(ids_d_ref, dec_emb, x_vmem, sem_g)
    wdma2.wait()
    _xproj(x_vmem, wih_buf, dbih_ref, dbhh_ref, xproj)
    _recurrence(xproj, whh_t, whh_tv, dh0, dc0, hs_ref)


def _out_kernel(hs_ref, w_ref, b_ref, o_ref):
    o_ref[...] = lax.dot_general(
        hs_ref[...], w_ref[...], _DNT,
        preferred_element_type=_F32) + b_ref[...]


_VB = 3200


def kernel(cond_emb, enc_emb, enc_wih, enc_whh, enc_bih, enc_bhh, mean_w,
           mean_b, lgv_w, lgv_b, l2h_w, l2h_b, l2c_w, l2c_b, dec_emb,
           dec_wih, dec_whh, dec_bih, dec_bhh, out_w, out_b, eps, input_ids,
           target_ids, input_c, target_c, *, interpret=False):
    ids_e = input_ids.astype(jnp.int32)
    ids_d = jnp.concatenate(
        [jnp.zeros((1,), jnp.int32), target_ids[:-1].astype(jnp.int32)])
    cond_i = cond_emb[input_c]
    cond_t = cond_emb[target_c].reshape(1, _C)
    h0 = jnp.concatenate([jnp.zeros((_H - _C,), _F32), cond_i]).reshape(1, _H)

    smem = pl.BlockSpec(memory_space=pltpu.SMEM)
    anyspace = pl.BlockSpec(memory_space=pl.ANY)
    vmem = pl.BlockSpec(memory_space=pltpu.VMEM)

    hs, m, lgv = pl.pallas_call(
        _seq_kernel,
        out_shape=[
            jax.ShapeDtypeStruct((_T, 1, _H), _F32),
            jax.ShapeDtypeStruct((1, _L), _F32),
            jax.ShapeDtypeStruct((1, _L), _F32),
        ],
        in_specs=[smem, smem] + [anyspace] * 6 + [vmem] * 15,
        out_specs=[vmem] * 3,
        scratch_shapes=[
            pltpu.VMEM((_T, _H), _F32),
            pltpu.VMEM((_T, 1, _G), _F32),
            pltpu.VMEM((_H, 3 * _H), _BF16),
            pltpu.VMEM((_H, _H), _F32),
            pltpu.VMEM((_G, _H), _F32),
            pltpu.VMEM((_G, _H), _F32),
            pltpu.SemaphoreType.DMA,
            pltpu.SemaphoreType.DMA,
            pltpu.SemaphoreType.DMA,
        ],
        compiler_params=pltpu.CompilerParams(
            vmem_limit_bytes=61 * 1024 * 1024),
        name="seq_lstm",
        interpret=interpret,
    )(ids_e, ids_d, enc_emb, dec_emb, enc_wih, enc_whh, dec_wih, dec_whh,
      enc_bih.reshape(1, _G), enc_bhh.reshape(1, _G),
      dec_bih.reshape(1, _G), dec_bhh.reshape(1, _G), h0,
      mean_w, mean_b.reshape(1, _L), lgv_w, lgv_b.reshape(1, _L),
      l2h_w, l2h_b.reshape(1, _H), l2c_w, l2c_b.reshape(1, _H),
      eps.reshape(1, _L), cond_t)
    hs2 = hs.reshape(_T, _H)

    logits = pl.pallas_call(
        _out_kernel,
        out_shape=jax.ShapeDtypeStruct((_T, _V), _F32),
        grid=(_V // _VB,),
        in_specs=[
            pl.BlockSpec((_T, _H), lambda i: (0, 0)),
            pl.BlockSpec((_VB, _H), lambda i: (i, 0)),
            pl.BlockSpec((1, _VB), lambda i: (0, i)),
        ],
        out_specs=pl.BlockSpec((_T, _VB), lambda i: (0, i)),
        compiler_params=pltpu.CompilerParams(
            dimension_semantics=("parallel",),
            vmem_limit_bytes=56 * 1024 * 1024),
        name="out_proj",
        interpret=interpret,
    )(hs2, out_w, out_b.reshape(1, _V))

    return logits, m.reshape(_L), lgv.reshape(_L)


# speedup vs baseline: 7.9430x; 1.0996x over previous
import jax
import jax.numpy as jnp
from jax import lax
from jax.experimental import pallas as pl
from jax.experimental.pallas import tpu as pltpu

_H = 1024
_C = 64
_L = 256
_V = 32000
_T = 256
_G = 4 * _H

_F32 = jnp.float32
_BF16 = jnp.bfloat16

_DNT = (((1,), (1,)), ((), ()))


def _lstm_gates(g_m, g_v, c):
    g_go = jnp.concatenate([g_m[:, 2 * _H:], g_v], axis=1)
    i = jax.nn.sigmoid(g_m[:, :_H])
    f = jax.nn.sigmoid(g_m[:, _H:2 * _H])
    gg = jnp.tanh(g_go[:, :_H])
    o = jax.nn.sigmoid(g_go[:, _H:])
    c = f * c + i * gg
    h = o * jnp.tanh(c)
    return h, c


def _start_gather(ids_ref, emb_ref, x_vmem, sem):
    for t in range(_T):
        pltpu.make_async_copy(emb_ref.at[ids_ref[t]], x_vmem.at[t], sem).start()


def _wait_gather(ids_ref, emb_ref, x_vmem, sem):
    for t in range(_T):
        pltpu.make_async_copy(emb_ref.at[ids_ref[t]], x_vmem.at[t], sem).wait()


_NM = 2560
_NV = _G - _NM


def _transpose_whh(whh_buf, whh_t, whh_tv):
    for j in range(_NM // 512):
        sl = slice(j * 512, (j + 1) * 512)
        whh_t[:, sl] = whh_buf[sl, :].T.astype(_BF16)
    for j in range(_NV // 512):
        src = slice(_NM + j * 512, _NM + (j + 1) * 512)
        whh_tv[:, j * 512:(j + 1) * 512] = whh_buf[src, :].T


def _xproj(x_vmem, wih_buf, bih_ref, bhh_ref, xproj):
    b = bih_ref[...] + bhh_ref[...]
    xp = lax.dot_general(x_vmem[...].astype(_BF16), wih_buf[...].astype(_BF16),
                         _DNT, preferred_element_type=_F32) + b
    xproj[...] = xp.reshape(_T, 1, _G)


def _recurrence(xproj, whh_t, whh_tv, h, c, hs_ref=None):
    def body(i, carry):
        h, c = carry
        t = i * 2
        w = whh_t[...]
        wv = whh_tv[...]
        for s in range(2):
            xg = xproj[t + s]
            g_m = xg[:, :_NM] + jnp.dot(h.astype(_BF16), w,
                                        preferred_element_type=_F32)
            h_col = h.reshape(1, _H).T
            g_v = xg[:, _NM:] + jnp.sum(h_col * wv, axis=0, keepdims=True)
            h, c = _lstm_gates(g_m, g_v, c)
            if hs_ref is not None:
                hs_ref[t + s] = h
        return (h, c)

    return lax.fori_loop(0, _T // 2, body, (h, c))


def _seq_kernel(ids_e_ref, ids_d_ref, enc_emb, dec_emb,
                enc_wih, enc_whh, dec_wih, dec_whh,
                ebih_ref, ebhh_ref, dbih_ref, dbhh_ref, h0_ref,
                mean_w_ref, mean_b_ref, lgv_w_ref, lgv_b_ref,
                l2h_w_ref, l2h_b_ref, l2c_w_ref, l2c_b_ref,
                eps_ref, cond_t_ref,
                hs_ref, m_ref, lgv_ref,
                x_vmem, xproj, whh_t, whh_tv, wih_buf, whh_buf,
                sem_g, sem_w, sem_h):
    wdma = pltpu.make_async_copy(enc_wih, wih_buf, sem_w)
    wdma.start()
    hdma = pltpu.make_async_copy(enc_whh, whh_buf, sem_h)
    hdma.start()
    _start_gather(ids_e_ref, enc_emb, x_vmem, sem_g)
    hdma.wait()
    _transpose_whh(whh_buf, whh_t, whh_tv)
    _wait_gather(ids_e_ref, enc_emb, x_vmem, sem_g)
    wdma.wait()
    _xproj(x_vmem, wih_buf, ebih_ref, ebhh_ref, xproj)

    wdma2 = pltpu.make_async_copy(dec_wih, wih_buf, sem_w)
    wdma2.start()
    hdma2 = pltpu.make_async_copy(dec_whh, whh_buf, sem_h)
    hdma2.start()
    _start_gather(ids_d_ref, dec_emb, x_vmem, sem_g)

    h0 = h0_ref[...]
    c0 = jnp.zeros((1, _H), _F32)
    h, _ = _recurrence(xproj, whh_t, whh_tv, h0, c0)

    m = lax.dot_general(h, mean_w_ref[...], _DNT,
                        preferred_element_type=_F32) + mean_b_ref[...]
    lgv = lax.dot_general(h, lgv_w_ref[...], _DNT,
                          preferred_element_type=_F32) + lgv_b_ref[...]
    z = eps_ref[...] * jnp.exp(lgv * 0.5) + m
    zc = jnp.concatenate([z, cond_t_ref[...]], axis=1)
    dh0 = lax.dot_general(zc, l2h_w_ref[...], _DNT,
                          preferred_element_type=_F32) + l2h_b_ref[...]
    dc0 = lax.dot_general(zc, l2c_w_ref[...], _DNT,
                          preferred_element_type=_F32) + l2c_b_ref[...]
    m_ref[...] = m
    lgv_ref[...] = lgv

    hdma2.wait()
    _transpose_whh(whh_buf, whh_t, whh_tv)
    _wait_gather(ids_d_ref, dec_emb, x_vmem, sem_g)
    wdma2.wait()
    _xproj(x_vmem, wih_buf, dbih_ref, dbhh_ref, xproj)
    _recurrence(xproj, whh_t, whh_tv, dh0, dc0, hs_ref)


def _out_kernel(hs_ref, w_ref, b_ref, o_ref):
    o_ref[...] = lax.dot_general(
        hs_ref[...], w_ref[...], _DNT,
        preferred_element_type=_F32) + b_ref[...]


_VB = 3200


def kernel(cond_emb, enc_emb, enc_wih, enc_whh, enc_bih, enc_bhh, mean_w,
           mean_b, lgv_w, lgv_b, l2h_w, l2h_b, l2c_w, l2c_b, dec_emb,
           dec_wih, dec_whh, dec_bih, dec_bhh, out_w, out_b, eps, input_ids,
           target_ids, input_c, target_c, *, interpret=False):
    ids_e = input_ids.astype(jnp.int32)
    ids_d = jnp.concatenate(
        [jnp.zeros((1,), jnp.int32), target_ids[:-1].astype(jnp.int32)])
    cond_i = cond_emb[input_c]
    cond_t = cond_emb[target_c].reshape(1, _C)
    h0 = jnp.concatenate([jnp.zeros((_H - _C,), _F32), cond_i]).reshape(1, _H)

    smem = pl.BlockSpec(memory_space=pltpu.SMEM)
    anyspace = pl.BlockSpec(memory_space=pl.ANY)
    vmem = pl.BlockSpec(memory_space=pltpu.VMEM)

    hs, m, lgv = pl.pallas_call(
        _seq_kernel,
        out_shape=[
            jax.ShapeDtypeStruct((_T, 1, _H), _F32),
            jax.ShapeDtypeStruct((1, _L), _F32),
            jax.ShapeDtypeStruct((1, _L), _F32),
        ],
        in_specs=[smem, smem] + [anyspace] * 6 + [vmem] * 15,
        out_specs=[vmem] * 3,
        scratch_shapes=[
            pltpu.VMEM((_T, _H), _F32),
            pltpu.VMEM((_T, 1, _G), _F32),
            pltpu.VMEM((_H, _NM), _BF16),
            pltpu.VMEM((_H, _NV), _F32),
            pltpu.VMEM((_G, _H), _F32),
            pltpu.VMEM((_G, _H), _F32),
            pltpu.SemaphoreType.DMA,
            pltpu.SemaphoreType.DMA,
            pltpu.SemaphoreType.DMA,
        ],
        compiler_params=pltpu.CompilerParams(
            vmem_limit_bytes=61 * 1024 * 1024),
        name="seq_lstm",
        interpret=interpret,
    )(ids_e, ids_d, enc_emb, dec_emb, enc_wih, enc_whh, dec_wih, dec_whh,
      enc_bih.reshape(1, _G), enc_bhh.reshape(1, _G),
      dec_bih.reshape(1, _G), dec_bhh.reshape(1, _G), h0,
      mean_w, mean_b.reshape(1, _L), lgv_w, lgv_b.reshape(1, _L),
      l2h_w, l2h_b.reshape(1, _H), l2c_w, l2c_b.reshape(1, _H),
      eps.reshape(1, _L), cond_t)
    hs2 = hs.reshape(_T, _H)

    logits = pl.pallas_call(
        _out_kernel,
        out_shape=jax.ShapeDtypeStruct((_T, _V), _F32),
        grid=(_V // _VB,),
        in_specs=[
            pl.BlockSpec((_T, _H), lambda i: (0, 0)),
            pl.BlockSpec((_VB, _H), lambda i: (i, 0)),
            pl.BlockSpec((1, _VB), lambda i: (0, i)),
        ],
        out_specs=pl.BlockSpec((_T, _VB), lambda i: (0, i)),
        compiler_params=pltpu.CompilerParams(
            dimension_semantics=("parallel",),
            vmem_limit_bytes=56 * 1024 * 1024),
        name="out_proj",
        interpret=interpret,
    )(hs2, out_w, out_b.reshape(1, _V))

    return logits, m.reshape(_L), lgv.reshape(_L)


# speedup vs baseline: 7.9456x; 1.0003x over previous
import jax
import jax.numpy as jnp
from jax import lax
from jax.experimental import pallas as pl
from jax.experimental.pallas import tpu as pltpu

_H = 1024
_C = 64
_L = 256
_V = 32000
_T = 256
_G = 4 * _H

_F32 = jnp.float32
_BF16 = jnp.bfloat16

_DNT = (((1,), (1,)), ((), ()))


def _lstm_gates(g_m, g_v, c):
    g_go = jnp.concatenate([g_m[:, 2 * _H:], g_v], axis=1)
    i = jax.nn.sigmoid(g_m[:, :_H])
    f = jax.nn.sigmoid(g_m[:, _H:2 * _H])
    gg = jnp.tanh(g_go[:, :_H])
    o = jax.nn.sigmoid(g_go[:, _H:])
    c = f * c + i * gg
    h = o * jnp.tanh(c)
    return h, c


def _start_gather(ids_ref, emb_ref, x_vmem, sem):
    for t in range(_T):
        pltpu.make_async_copy(emb_ref.at[ids_ref[t]], x_vmem.at[t], sem).start()


def _wait_gather(ids_ref, emb_ref, x_vmem, sem):
    for t in range(_T):
        pltpu.make_async_copy(emb_ref.at[ids_ref[t]], x_vmem.at[t], sem).wait()


_NM = 2560
_NV = _G - _NM


def _transpose_whh(whh_buf, whh_t, whh_tv):
    for j in range(_NM // 512):
        sl = slice(j * 512, (j + 1) * 512)
        whh_t[:, sl] = whh_buf[sl, :].T.astype(_BF16)
    for j in range(_NV // 512):
        src = slice(_NM + j * 512, _NM + (j + 1) * 512)
        whh_tv[:, j * 512:(j + 1) * 512] = whh_buf[src, :].T


def _xproj(x_vmem, wih_buf, bih_ref, bhh_ref, xproj):
    b = bih_ref[...] + bhh_ref[...]
    xp = lax.dot_general(x_vmem[...].astype(_BF16), wih_buf[...].astype(_BF16),
                         _DNT, preferred_element_type=_F32) + b
    xproj[...] = xp.reshape(_T, 1, _G)


def _recurrence(xproj, whh_t, whh_tv, h, c, hs_ref=None):
    def body(i, carry):
        h, c = carry
        t = i * 2
        w = whh_t[...]
        wv = whh_tv[...]
        for s in range(2):
            xg = xproj[t + s]
            g_m = xg[:, :_NM] + jnp.dot(h.astype(_BF16), w,
                                        preferred_element_type=_F32)
            h_col = h.reshape(1, _H).T
            g_v = xg[:, _NM:] + jnp.sum(h_col * wv, axis=0, keepdims=True)
            h, c = _lstm_gates(g_m, g_v, c)
            if hs_ref is not None:
                hs_ref[t + s] = h
        return (h, c)

    return lax.fori_loop(0, _T // 2, body, (h, c))


def _seq_kernel(ids_e_ref, ids_d_ref, enc_emb, dec_emb,
                enc_wih, enc_whh, dec_wih, dec_whh,
                ebih_ref, ebhh_ref, dbih_ref, dbhh_ref, h0_ref,
                mean_w_ref, mean_b_ref, lgv_w_ref, lgv_b_ref,
                l2h_w_ref, l2h_b_ref, l2c_w_ref, l2c_b_ref,
                eps_ref, cond_t_ref,
                hs_ref, m_ref, lgv_ref,
                x_vmem, xproj, whh_t, whh_tv, wih_buf, whh_buf,
                sem_g, sem_w, sem_h):
    wdma = pltpu.make_async_copy(enc_wih, wih_buf, sem_w)
    wdma.start()
    hdma = pltpu.make_async_copy(enc_whh, whh_buf, sem_h)
    hdma.start()
    _start_gather(ids_e_ref, enc_emb, x_vmem, sem_g)
    hdma.wait()
    _transpose_whh(whh_buf, whh_t, whh_tv)
    _wait_gather(ids_e_ref, enc_emb, x_vmem, sem_g)
    wdma.wait()
    _xproj(x_vmem, wih_buf, ebih_ref, ebhh_ref, xproj)

    wdma2 = pltpu.make_async_copy(dec_wih, wih_buf, sem_w)
    wdma2.start()
    hdma2 = pltpu.make_async_copy(dec_whh, whh_buf, sem_h)
    hdma2.start()
    _start_gather(ids_d_ref, dec_emb, x_vmem, sem_g)

    h0 = h0_ref[...]
    c0 = jnp.zeros((1, _H), _F32)
    h, _ = _recurrence(xproj, whh_t, whh_tv, h0, c0)

    m = lax.dot_general(h, mean_w_ref[...], _DNT,
                        preferred_element_type=_F32) + mean_b_ref[...]
    lgv = lax.dot_general(h, lgv_w_ref[...], _DNT,
                          preferred_element_type=_F32) + lgv_b_ref[...]
    z = eps_ref[...] * jnp.exp(lgv * 0.5) + m
    zc = jnp.concatenate([z, cond_t_ref[...]], axis=1)
    dh0 = lax.dot_general(zc, l2h_w_ref[...], _DNT,
                          preferred_element_type=_F32) + l2h_b_ref[...]
    dc0 = lax.dot_general(zc, l2c_w_ref[...], _DNT,
                          preferred_element_type=_F32) + l2c_b_ref[...]
    m_ref[...] = m
    lgv_ref[...] = lgv

    hdma2.wait()
    _transpose_whh(whh_buf, whh_t, whh_tv)
    _wait_gather(ids_d_ref, dec_emb, x_vmem, sem_g)
    wdma2.wait()
    _xproj(x_vmem, wih_buf, dbih_ref, dbhh_ref, xproj)
    _recurrence(xproj, whh_t, whh_tv, dh0, dc0, hs_ref)


def _out_kernel(hs_ref, w_ref, b_ref, o_ref):
    o_ref[...] = lax.dot_general(
        hs_ref[...], w_ref[...], _DNT,
        preferred_element_type=_F32) + b_ref[...]


_VB = 3200


def kernel(cond_emb, enc_emb, enc_wih, enc_whh, enc_bih, enc_bhh, mean_w,
           mean_b, lgv_w, lgv_b, l2h_w, l2h_b, l2c_w, l2c_b, dec_emb,
           dec_wih, dec_whh, dec_bih, dec_bhh, out_w, out_b, eps, input_ids,
           target_ids, input_c, target_c):
    ids_e = input_ids.astype(jnp.int32)
    ids_d = jnp.concatenate(
        [jnp.zeros((1,), jnp.int32), target_ids[:-1].astype(jnp.int32)])
    cond_i = cond_emb[input_c]
    cond_t = cond_emb[target_c].reshape(1, _C)
    h0 = jnp.concatenate([jnp.zeros((_H - _C,), _F32), cond_i]).reshape(1, _H)

    smem = pl.BlockSpec(memory_space=pltpu.SMEM)
    anyspace = pl.BlockSpec(memory_space=pl.ANY)
    vmem = pl.BlockSpec(memory_space=pltpu.VMEM)

    hs, m, lgv = pl.pallas_call(
        _seq_kernel,
        out_shape=[
            jax.ShapeDtypeStruct((_T, 1, _H), _F32),
            jax.ShapeDtypeStruct((1, _L), _F32),
            jax.ShapeDtypeStruct((1, _L), _F32),
        ],
        in_specs=[smem, smem] + [anyspace] * 6 + [vmem] * 15,
        out_specs=[vmem] * 3,
        scratch_shapes=[
            pltpu.VMEM((_T, _H), _F32),
            pltpu.VMEM((_T, 1, _G), _F32),
            pltpu.VMEM((_H, _NM), _BF16),
            pltpu.VMEM((_H, _NV), _F32),
            pltpu.VMEM((_G, _H), _F32),
            pltpu.VMEM((_G, _H), _F32),
            pltpu.SemaphoreType.DMA,
            pltpu.SemaphoreType.DMA,
            pltpu.SemaphoreType.DMA,
        ],
        compiler_params=pltpu.CompilerParams(
            vmem_limit_bytes=61 * 1024 * 1024),
        name="seq_lstm",
    )(ids_e, ids_d, enc_emb, dec_emb, enc_wih, enc_whh, dec_wih, dec_whh,
      enc_bih.reshape(1, _G), enc_bhh.reshape(1, _G),
      dec_bih.reshape(1, _G), dec_bhh.reshape(1, _G), h0,
      mean_w, mean_b.reshape(1, _L), lgv_w, lgv_b.reshape(1, _L),
      l2h_w, l2h_b.reshape(1, _H), l2c_w, l2c_b.reshape(1, _H),
      eps.reshape(1, _L), cond_t)
    hs2 = hs.reshape(_T, _H)

    logits = pl.pallas_call(
        _out_kernel,
        out_shape=jax.ShapeDtypeStruct((_T, _V), _F32),
        grid=(_V // _VB,),
        in_specs=[
            pl.BlockSpec((_T, _H), lambda i: (0, 0)),
            pl.BlockSpec((_VB, _H), lambda i: (i, 0)),
            pl.BlockSpec((1, _VB), lambda i: (0, i)),
        ],
        out_specs=pl.BlockSpec((_T, _VB), lambda i: (0, i)),
        compiler_params=pltpu.CompilerParams(
            dimension_semantics=("parallel",),
            vmem_limit_bytes=56 * 1024 * 1024),
        name="out_proj",
    )(hs2, out_w, out_b.reshape(1, _V))

    return logits, m.reshape(_L), lgv.reshape(_L)
